# baseline (speedup 1.0000x reference)
import jax
import jax.numpy as jnp
from jax import lax
from jax.experimental import pallas as pl
from jax.experimental.pallas import tpu as pltpu
from jax.experimental.pallas import tpu_sc as plsc

VOCAB_SIZE = 1000000
EMBED_DIM = 64
BATCH = 4
SEQ_LEN = 8192
SCALE = 8.0

NUM_CORES = 2
NUM_SUBCORES = 16
NUM_WORKERS = NUM_CORES * NUM_SUBCORES
TOTAL = BATCH * SEQ_LEN
B_PER_W = TOTAL // NUM_WORKERS
LANES = 16

N_BLOCKS = 7812
STEP_W = 4
N_STEPS = N_BLOCKS // STEP_W
STEPS_W0 = 62
STEPS_W = 61
TAIL_START = N_BLOCKS * 128
T2_ROWS = N_BLOCKS * 64

CHUNK = 256
N_CHUNKS = B_PER_W // CHUNK
GROUPS = CHUNK // LANES


def _transpose_block(blk, out_v, lane):
    def diag(v0, carry):
        u_vec = (lane + v0) & 127
        row_half = u_vec >> 1
        col_base = (u_vec & 1) << 6
        for s in range(STEP_W):
            for k in range(EMBED_DIM // LANES):
                c_vec = lane + k * LANES
                vals = plsc.load_gather(blk, [c_vec, s * 128 + u_vec])
                plsc.store_scatter(
                    out_v, [s * 64 + row_half, col_base + c_vec], vals * SCALE
                )
        return carry

    lax.fori_loop(0, 128, diag, 0)


def _body_a(tT_hbm, t2_hbm, blk0, blk1, out_v, sem0, sem1, semo):
    wid = lax.axis_index("s") * NUM_CORES + lax.axis_index("c")
    lane = lax.iota(jnp.int32, LANES)
    my_lo = STEPS_W * wid + jnp.minimum(wid, 1)
    my_n = jnp.where(wid == 0, STEPS_W0, STEPS_W)

    def in_copy(step, buf, sem):
        col0 = pl.multiple_of(step * (STEP_W * 128), STEP_W * 128)
        return pltpu.async_copy(tT_hbm.at[:, pl.ds(col0, STEP_W * 128)], buf, sem)

    def out_copy(step):
        row0 = pl.multiple_of(step * (STEP_W * 64), STEP_W * 64)
        return pltpu.async_copy(out_v, t2_hbm.at[pl.ds(row0, STEP_W * 64)], semo)

    @pl.when(my_n > 0)
    def _():
        in_copy(my_lo, blk0, sem0).wait()
        _transpose_block(blk0, out_v, lane)
        out_copy(my_lo).wait()

    def step_pair(t, carry):
        i1 = my_lo + 2 * t + 1
        i2 = my_lo + 2 * t + 2

        @pl.when(2 * t + 1 < my_n)
        def _():
            cp1 = in_copy(i1, blk1, sem1)

            @pl.when(2 * t + 2 < my_n)
            def _():
                in_copy(i2, blk0, sem0)

            cp1.wait()
            _transpose_block(blk1, out_v, lane)
            out_copy(i1).wait()

        @pl.when(2 * t + 2 < my_n)
        def _():
            pltpu.make_async_copy(
                tT_hbm.at[:, pl.ds(0, STEP_W * 128)], blk0, sem0
            ).wait()
            _transpose_block(blk0, out_v, lane)
            out_copy(i2).wait()

        return carry

    lax.fori_loop(0, (STEPS_W0 + 1) // 2, step_pair, 0)


def _body_b(t2_hbm, idx_hbm, tail_hbm, out_hbm, idx_v, idx2_v, tail_v, rows_v, out_v, sem):
    wid = lax.axis_index("s") * NUM_CORES + lax.axis_index("c")
    base = wid * B_PER_W
    lane = lax.iota(jnp.int32, LANES)
    pltpu.sync_copy(tail_hbm, tail_v)

    def chunk_body(q, carry):
        cbase = base + q * CHUNK
        pltpu.sync_copy(idx_hbm.at[pl.ds(cbase, CHUNK)], idx_v)

        def pairify(k, c2):
            v = idx_v[pl.ds(k * LANES, LANES)]
            idx2_v[pl.ds(k * LANES, LANES)] = jnp.minimum(v, TAIL_START - 1) >> 1
            return c2

        lax.fori_loop(0, GROUPS, pairify, 0)
        pltpu.async_copy(t2_hbm.at[idx2_v], rows_v, sem).wait()

        def select_group(g, c2):
            t_vec = lane + g * LANES
            idxg = idx_v[pl.ds(g * LANES, LANES)]
            par = (idxg & 1) << 6
            tmask = idxg >= TAIL_START
            trow = jnp.where(tmask, idxg - TAIL_START, 0)
            for c in range(EMBED_DIM):
                cols = (lane + c) & (EMBED_DIM - 1)
                vals = plsc.load_gather(rows_v, [t_vec, par + cols])
                tvals = plsc.load_gather(tail_v, [trow, cols])
                sel = jnp.where(tmask, tvals, vals)
                plsc.store_scatter(out_v, [t_vec, cols], sel)
            return c2

        lax.fori_loop(0, GROUPS, select_group, 0)
        pltpu.sync_copy(out_v, out_hbm.at[pl.ds(cbase, CHUNK)])
        return carry

    lax.fori_loop(0, N_CHUNKS, chunk_body, 0)


@jax.jit
def _embed(tT, idx, tail):
    mesh = plsc.VectorSubcoreMesh(core_axis_name="c", subcore_axis_name="s")
    params = pltpu.CompilerParams(use_tc_tiling_on_sc=True, needs_layout_passes=False)
    t2 = pl.kernel(
        _body_a,
        out_type=jax.ShapeDtypeStruct((T2_ROWS, 2 * EMBED_DIM), jnp.float32),
        mesh=mesh,
        scratch_types=[
            pltpu.VMEM((EMBED_DIM, STEP_W * 128), jnp.float32),
            pltpu.VMEM((EMBED_DIM, STEP_W * 128), jnp.float32),
            pltpu.VMEM((STEP_W * 64, 2 * EMBED_DIM), jnp.float32),
            pltpu.SemaphoreType.DMA,
            pltpu.SemaphoreType.DMA,
            pltpu.SemaphoreType.DMA,
        ],
        compiler_params=params,
    )(tT)
    out = pl.kernel(
        _body_b,
        out_type=jax.ShapeDtypeStruct((TOTAL, EMBED_DIM), jnp.float32),
        mesh=mesh,
        scratch_types=[
            pltpu.VMEM((CHUNK,), jnp.int32),
            pltpu.VMEM((CHUNK,), jnp.int32),
            pltpu.VMEM((EMBED_DIM, EMBED_DIM), jnp.float32),
            pltpu.VMEM((CHUNK, 2 * EMBED_DIM), jnp.float32),
            pltpu.VMEM((CHUNK, EMBED_DIM), jnp.float32),
            pltpu.SemaphoreType.DMA,
        ],
        compiler_params=params,
    )(t2, idx, tail)
    return out


def kernel(x, input_embedding_table):
    idx = x.reshape(-1).astype(jnp.int32)
    tail = input_embedding_table[TAIL_START:] * SCALE
    out = _embed(input_embedding_table.T, idx, tail)
    return out.reshape(BATCH, SEQ_LEN, EMBED_DIM)


# speedup vs baseline: 1.3905x; 1.3905x over previous
import functools

import jax
import jax.numpy as jnp
from jax import lax
from jax.experimental import pallas as pl
from jax.experimental.pallas import tpu as pltpu
from jax.experimental.pallas import tpu_sc as plsc

VOCAB_SIZE = 1000000
EMBED_DIM = 64
BATCH = 4
SEQ_LEN = 8192
SCALE = 8.0

NUM_CORES = 2
NUM_SUBCORES = 16
NUM_WORKERS = NUM_CORES * NUM_SUBCORES
TOTAL = BATCH * SEQ_LEN
B_PER_W = TOTAL // NUM_WORKERS
LANES = 16


def _body(table_hbm, idx_hbm, out_hbm, idx_v, rows_v, sem):
    wid = lax.axis_index("s") * NUM_CORES + lax.axis_index("c")
    base = wid * B_PER_W
    pltpu.sync_copy(idx_hbm.at[pl.ds(base, B_PER_W)], idx_v)
    pltpu.async_copy(table_hbm.at[idx_v], rows_v, sem).wait()

    def scale_row(i, carry):
        for j in range(EMBED_DIM // LANES):
            sl = rows_v[i, pl.ds(j * LANES, LANES)]
            rows_v[i, pl.ds(j * LANES, LANES)] = sl * SCALE
        return carry

    lax.fori_loop(0, B_PER_W, scale_row, 0)
    pltpu.sync_copy(rows_v, out_hbm.at[pl.ds(base, B_PER_W)])


@jax.jit
def _embed(table, idx):
    mesh = plsc.VectorSubcoreMesh(core_axis_name="c", subcore_axis_name="s")
    run = pl.kernel(
        _body,
        out_type=jax.ShapeDtypeStruct((TOTAL, EMBED_DIM), jnp.float32),
        mesh=mesh,
        scratch_types=[
            pltpu.VMEM((B_PER_W,), jnp.int32),
            pltpu.VMEM((B_PER_W, EMBED_DIM), jnp.float32),
            pltpu.SemaphoreType.DMA,
        ],
        compiler_params=pltpu.CompilerParams(use_tc_tiling_on_sc=False),
    )
    return run(table, idx)


def kernel(x, input_embedding_table):
    idx = x.reshape(-1).astype(jnp.int32)
    out = _embed(input_embedding_table, idx)
    return out.reshape(BATCH, SEQ_LEN, EMBED_DIM)


# speedup vs baseline: 1.3923x; 1.0013x over previous
import functools

import jax
import jax.numpy as jnp
from jax import lax
from jax.experimental import pallas as pl
from jax.experimental.pallas import tpu as pltpu
from jax.experimental.pallas import tpu_sc as plsc

VOCAB_SIZE = 1000000
EMBED_DIM = 64
BATCH = 4
SEQ_LEN = 8192
SCALE = 8.0

NUM_CORES = 2
NUM_SUBCORES = 16
NUM_WORKERS = NUM_CORES * NUM_SUBCORES
TOTAL = BATCH * SEQ_LEN
B_PER_W = TOTAL // NUM_WORKERS
LANES = 16


def _body(table_hbm, idx_hbm, out_hbm, idx_v, rows_v, sem):
    wid = lax.axis_index("s") * NUM_CORES + lax.axis_index("c")
    base = wid * B_PER_W
    pltpu.sync_copy(idx_hbm.at[pl.ds(base, B_PER_W)], idx_v)
    pltpu.async_copy(table_hbm.at[idx_v], rows_v, sem).wait()

    def scale_row(i, carry):
        for j in range(EMBED_DIM // LANES):
            sl = rows_v[i, pl.ds(j * LANES, LANES)]
            rows_v[i, pl.ds(j * LANES, LANES)] = sl * SCALE
        return carry

    lax.fori_loop(0, B_PER_W, scale_row, 0)
    w_per_b = SEQ_LEN // B_PER_W
    b = wid // w_per_b
    s0 = (wid % w_per_b) * B_PER_W
    pltpu.sync_copy(rows_v, out_hbm.at[b, pl.ds(s0, B_PER_W)])


@jax.jit
def _embed(table, idx):
    mesh = plsc.VectorSubcoreMesh(core_axis_name="c", subcore_axis_name="s")
    run = pl.kernel(
        _body,
        out_type=jax.ShapeDtypeStruct((BATCH, SEQ_LEN, EMBED_DIM), jnp.float32),
        mesh=mesh,
        scratch_types=[
            pltpu.VMEM((B_PER_W,), jnp.int32),
            pltpu.VMEM((B_PER_W, EMBED_DIM), jnp.float32),
            pltpu.SemaphoreType.DMA,
        ],
        compiler_params=pltpu.CompilerParams(use_tc_tiling_on_sc=False),
    )
    return run(table, idx)


def kernel(x, input_embedding_table):
    idx = x.reshape(-1).astype(jnp.int32)
    return _embed(input_embedding_table, idx)
